# baseline (speedup 1.0000x reference)
import jax
import jax.numpy as jnp
from jax.experimental import pallas as pl
from jax.experimental.pallas import tpu as pltpu


def _plnet_body(x_ref, o1_ref, o2_ref, o3_ref, o4_ref):
    x = x_ref[0]

    def terms(base):
        a = x[base : base + 1, :] * x[base + 1 : base + 21, :]
        gx = x[base + 23 : base + 37, :]
        gy = x[base + 37 : base + 51, :]
        return a, gx, gy

    A1, cx1, cy1 = terms(0)
    A2, cx2, cy2 = terms(51)
    B1, zx1, zy1 = terms(102)
    B2, zx2, zy2 = terms(153)

    s_row = jax.lax.broadcasted_iota(jnp.int32, (14, 196), 0)
    p_col = jax.lax.broadcasted_iota(jnp.int32, (14, 196), 1)
    Rt = (p_col // 14 == s_row).astype(jnp.float32)
    Tt = (p_col % 14 == s_row).astype(jnp.float32)

    dn = (((0,), (0,)), ((), ()))

    def dotT(a, b):
        return jax.lax.dot_general(
            a, b, dn,
            preferred_element_type=jnp.float32,
            precision=jax.lax.Precision.HIGHEST,
        )

    Lc1 = dotT(cx1, Rt) * dotT(cy1, Tt)
    Lc2 = dotT(cx2, Rt) * dotT(cy2, Tt)
    Lz1 = dotT(Rt, zx1) * dotT(Tt, zy1)
    Lz2 = dotT(Rt, zx2) * dotT(Tt, zy2)

    def emit(o_ref, A, B, Lc, Lz):
        L = (0.5 * Lc) * Lz
        t = (A[:, :, None] * L[None, :, :]) * B[:, None, :]
        o_ref[0] = t

    emit(o1_ref, A1, B1, Lc1, Lz1)
    emit(o2_ref, A2, B1, Lc2, Lz1)
    emit(o3_ref, A1, B2, Lc1, Lz2)
    emit(o4_ref, A2, B2, Lc2, Lz2)


def kernel(inference):
    N = inference.shape[0]
    inf = inference.reshape(N, 204, 196)
    out_sds = jax.ShapeDtypeStruct((N, 20, 196, 196), jnp.float32)
    outs = pl.pallas_call(
        _plnet_body,
        grid=(N,),
        in_specs=[pl.BlockSpec((1, 204, 196), lambda n: (n, 0, 0))],
        out_specs=[pl.BlockSpec((1, 20, 196, 196), lambda n: (n, 0, 0, 0))] * 4,
        out_shape=[out_sds] * 4,
        compiler_params=pltpu.CompilerParams(
            dimension_semantics=("parallel",),
        ),
    )(inf)
    return tuple(o.reshape(N, 20, 14, 14, 14, 14) for o in outs)


# speedup vs baseline: 1.8159x; 1.8159x over previous
import jax
import jax.numpy as jnp
from jax.experimental import pallas as pl
from jax.experimental.pallas import tpu as pltpu


def _prep_body(x_ref, a1_ref, a2_ref, b1_ref, b2_ref,
               cx1_ref, cy1_ref, cx2_ref, cy2_ref,
               zx1_ref, zy1_ref, zx2_ref, zy2_ref):
    x = x_ref[...]

    def cls(base):
        return x[:, base : base + 1, :] * x[:, base + 1 : base + 21, :]

    a1_ref[...] = jnp.transpose(cls(0), (2, 0, 1))
    a2_ref[...] = jnp.transpose(cls(51), (2, 0, 1))
    cx1_ref[...] = jnp.transpose(x[:, 23:37, :], (2, 0, 1))
    cy1_ref[...] = jnp.transpose(x[:, 37:51, :], (2, 0, 1))
    cx2_ref[...] = jnp.transpose(x[:, 74:88, :], (2, 0, 1))
    cy2_ref[...] = jnp.transpose(x[:, 88:102, :], (2, 0, 1))
    b1_ref[...] = jnp.transpose(0.5 * cls(102), (2, 0, 1))
    b2_ref[...] = jnp.transpose(0.5 * cls(153), (2, 0, 1))
    zx1_ref[...] = jnp.transpose(x[:, 125:139, :], (1, 0, 2))
    zy1_ref[...] = jnp.transpose(x[:, 139:153, :], (1, 0, 2))
    zx2_ref[...] = jnp.transpose(x[:, 176:190, :], (1, 0, 2))
    zy2_ref[...] = jnp.transpose(x[:, 190:204, :], (1, 0, 2))


def _big_body(a1_ref, a2_ref, b1_ref, b2_ref,
              cx1_ref, cy1_ref, cx2_ref, cy2_ref,
              zx1_ref, zy1_ref, zx2_ref, zy2_ref,
              o1_ref, o2_ref, o3_ref, o4_ref):
    s_row = jax.lax.broadcasted_iota(jnp.int32, (14, 196), 0)
    p_col = jax.lax.broadcasted_iota(jnp.int32, (14, 196), 1)
    Rt = (p_col // 14 == s_row).astype(jnp.float32)
    Tt = (p_col % 14 == s_row).astype(jnp.float32)

    def sel(slab, onehot):
        return jax.lax.dot_general(
            slab, onehot, (((1,), (0,)), ((), ())),
            preferred_element_type=jnp.float32,
            precision=jax.lax.Precision.HIGHEST,
        )

    Lc1 = sel(cx1_ref[0], Rt) * sel(cy1_ref[0], Tt)
    Lc2 = sel(cx2_ref[0], Rt) * sel(cy2_ref[0], Tt)
    Lz1 = zx1_ref[0] * zy1_ref[0]
    Lz2 = zx2_ref[0] * zy2_ref[0]

    A1 = a1_ref[0]
    A2 = a2_ref[0]
    B1 = b1_ref[...]
    B2 = b2_ref[...]

    def emit(o_ref, W, A, B):
        WT = jnp.transpose(W)
        o_ref[0] = (WT[:, :, None] * A[None, :, :]) * B

    emit(o1_ref, Lc1 * Lz1, A1, B1)
    emit(o2_ref, Lc2 * Lz1, A2, B1)
    emit(o3_ref, Lc1 * Lz2, A1, B2)
    emit(o4_ref, Lc2 * Lz2, A2, B2)


def kernel(inference):
    N = inference.shape[0]
    inf = inference.reshape(N, 204, 196)
    f32 = jnp.float32
    prep = pl.pallas_call(
        _prep_body,
        out_shape=[jax.ShapeDtypeStruct((196, 16, 20), f32)] * 4
        + [jax.ShapeDtypeStruct((196, 16, 14), f32)] * 4
        + [jax.ShapeDtypeStruct((14, 16, 196), f32)] * 4,
    )(inf)
    a1, a2, b1, b2, cx1, cy1, cx2, cy2, zx1, zy1, zx2, zy2 = prep

    slab20 = pl.BlockSpec((1, 16, 20), lambda i: (i, 0, 0))
    slab14 = pl.BlockSpec((1, 16, 14), lambda i: (i, 0, 0))
    full20 = pl.BlockSpec((196, 16, 20), lambda i: (0, 0, 0))
    rowx = pl.BlockSpec((1, 16, 196), lambda i: (i // 14, 0, 0))
    rowy = pl.BlockSpec((1, 16, 196), lambda i: (i % 14, 0, 0))
    outs = pl.pallas_call(
        _big_body,
        grid=(196,),
        in_specs=[slab20, slab20, full20, full20,
                  slab14, slab14, slab14, slab14,
                  rowx, rowy, rowx, rowy],
        out_specs=[pl.BlockSpec((1, 196, 16, 20), lambda i: (i, 0, 0, 0))] * 4,
        out_shape=[jax.ShapeDtypeStruct((196, 196, 16, 20), f32)] * 4,
        compiler_params=pltpu.CompilerParams(
            dimension_semantics=("parallel",),
        ),
    )(a1, a2, b1, b2, cx1, cy1, cx2, cy2, zx1, zy1, zx2, zy2)
    return tuple(
        jnp.transpose(o, (2, 3, 0, 1)).reshape(N, 20, 14, 14, 14, 14) for o in outs
    )


# speedup vs baseline: 1.9022x; 1.0476x over previous
import jax
import jax.numpy as jnp
from jax.experimental import pallas as pl
from jax.experimental.pallas import tpu as pltpu


def _prep_body(x_ref, a1_ref, a2_ref, b1_ref, b2_ref,
               cx1_ref, cy1_ref, cx2_ref, cy2_ref,
               zx1_ref, zy1_ref, zx2_ref, zy2_ref):
    x = x_ref[...]

    def cls(base):
        return x[:, base : base + 1, :] * x[:, base + 1 : base + 21, :]

    a1_ref[...] = jnp.transpose(cls(0), (2, 0, 1))
    a2_ref[...] = jnp.transpose(cls(51), (2, 0, 1))
    cx1_ref[...] = jnp.transpose(x[:, 23:37, :], (2, 0, 1))
    cy1_ref[...] = jnp.transpose(x[:, 37:51, :], (2, 0, 1))
    cx2_ref[...] = jnp.transpose(x[:, 74:88, :], (2, 0, 1))
    cy2_ref[...] = jnp.transpose(x[:, 88:102, :], (2, 0, 1))
    b1_ref[...] = jnp.transpose(0.5 * cls(102), (2, 0, 1))
    b2_ref[...] = jnp.transpose(0.5 * cls(153), (2, 0, 1))
    zx1_ref[...] = jnp.transpose(x[:, 125:139, :], (1, 0, 2))
    zy1_ref[...] = jnp.transpose(x[:, 139:153, :], (1, 0, 2))
    zx2_ref[...] = jnp.transpose(x[:, 176:190, :], (1, 0, 2))
    zy2_ref[...] = jnp.transpose(x[:, 190:204, :], (1, 0, 2))


_IB = 4


def _big_body(a1_ref, a2_ref, b1_ref, b2_ref,
              cx1_ref, cy1_ref, cx2_ref, cy2_ref,
              zx1_ref, zy1_ref, zx2_ref, zy2_ref,
              o1_ref, o2_ref, o3_ref, o4_ref):
    s_row = jax.lax.broadcasted_iota(jnp.int32, (14, 196), 0)
    p_col = jax.lax.broadcasted_iota(jnp.int32, (14, 196), 1)
    Rt = (p_col // 14 == s_row).astype(jnp.float32)
    Tt = (p_col % 14 == s_row).astype(jnp.float32)

    def sel(slab, onehot):
        return jax.lax.dot_general(
            slab, onehot, (((1,), (0,)), ((), ())),
            preferred_element_type=jnp.float32,
            precision=jax.lax.Precision.HIGHEST,
        )

    Lc1 = sel(cx1_ref[...].reshape(16 * _IB, 14), Rt) * sel(
        cy1_ref[...].reshape(16 * _IB, 14), Tt)
    Lc2 = sel(cx2_ref[...].reshape(16 * _IB, 14), Rt) * sel(
        cy2_ref[...].reshape(16 * _IB, 14), Tt)

    B1 = b1_ref[...]
    B2 = b2_ref[...]

    i0 = pl.program_id(0) * _IB
    for k in range(_IB):
        i = i0 + k
        ix = jax.lax.div(i, 14)
        iy = jax.lax.rem(i, 14)
        Lz1 = zx1_ref[ix] * zy1_ref[iy]
        Lz2 = zx2_ref[ix] * zy2_ref[iy]
        lo, hi = 16 * k, 16 * (k + 1)
        Lc1k = Lc1[lo:hi]
        Lc2k = Lc2[lo:hi]
        A1 = a1_ref[k]
        A2 = a2_ref[k]

        def emit(o_ref, W, A, B):
            WT = jnp.transpose(W)
            o_ref[k] = (WT[:, :, None] * A[None, :, :]) * B

        emit(o1_ref, Lc1k * Lz1, A1, B1)
        emit(o2_ref, Lc2k * Lz1, A2, B1)
        emit(o3_ref, Lc1k * Lz2, A1, B2)
        emit(o4_ref, Lc2k * Lz2, A2, B2)


def kernel(inference):
    N = inference.shape[0]
    inf = inference.reshape(N, 204, 196)
    f32 = jnp.float32
    prep = pl.pallas_call(
        _prep_body,
        out_shape=[jax.ShapeDtypeStruct((196, 16, 20), f32)] * 4
        + [jax.ShapeDtypeStruct((196, 16, 14), f32)] * 4
        + [jax.ShapeDtypeStruct((14, 16, 196), f32)] * 4,
    )(inf)
    a1, a2, b1, b2, cx1, cy1, cx2, cy2, zx1, zy1, zx2, zy2 = prep

    slab20 = pl.BlockSpec((_IB, 16, 20), lambda i: (i, 0, 0))
    slab14 = pl.BlockSpec((_IB, 16, 14), lambda i: (i, 0, 0))
    full20 = pl.BlockSpec((196, 16, 20), lambda i: (0, 0, 0))
    fullz = pl.BlockSpec((14, 16, 196), lambda i: (0, 0, 0))
    outs = pl.pallas_call(
        _big_body,
        grid=(196 // _IB,),
        in_specs=[slab20, slab20, full20, full20,
                  slab14, slab14, slab14, slab14,
                  fullz, fullz, fullz, fullz],
        out_specs=[pl.BlockSpec((_IB, 196, 16, 20), lambda i: (i, 0, 0, 0))] * 4,
        out_shape=[jax.ShapeDtypeStruct((196, 196, 16, 20), f32)] * 4,
        compiler_params=pltpu.CompilerParams(
            dimension_semantics=("parallel",),
        ),
    )(a1, a2, b1, b2, cx1, cy1, cx2, cy2, zx1, zy1, zx2, zy2)
    return tuple(
        jnp.transpose(o, (2, 3, 0, 1)).reshape(N, 20, 14, 14, 14, 14) for o in outs
    )


# speedup vs baseline: 1.9050x; 1.0014x over previous
import jax
import jax.numpy as jnp
from jax.experimental import pallas as pl
from jax.experimental.pallas import tpu as pltpu

_IB = 2


def _fused_body(x_ref, o1_ref, o2_ref, o3_ref, o4_ref,
                a1_s, a2_s, b1_s, b2_s,
                cx1_s, cy1_s, cx2_s, cy2_s,
                zx1_s, zy1_s, zx2_s, zy2_s):
    @pl.when(pl.program_id(0) == 0)
    def _prep():
        x = x_ref[...]

        def cls(base):
            return x[:, base : base + 1, :] * x[:, base + 1 : base + 21, :]

        a1_s[...] = jnp.transpose(cls(0), (2, 0, 1))
        a2_s[...] = jnp.transpose(cls(51), (2, 0, 1))
        cx1_s[...] = jnp.transpose(x[:, 23:37, :], (2, 0, 1))
        cy1_s[...] = jnp.transpose(x[:, 37:51, :], (2, 0, 1))
        cx2_s[...] = jnp.transpose(x[:, 74:88, :], (2, 0, 1))
        cy2_s[...] = jnp.transpose(x[:, 88:102, :], (2, 0, 1))
        b1_s[...] = jnp.transpose(0.5 * cls(102), (2, 0, 1))
        b2_s[...] = jnp.transpose(0.5 * cls(153), (2, 0, 1))
        zx1_s[...] = jnp.transpose(x[:, 125:139, :], (1, 0, 2))
        zy1_s[...] = jnp.transpose(x[:, 139:153, :], (1, 0, 2))
        zx2_s[...] = jnp.transpose(x[:, 176:190, :], (1, 0, 2))
        zy2_s[...] = jnp.transpose(x[:, 190:204, :], (1, 0, 2))

    s_row = jax.lax.broadcasted_iota(jnp.int32, (14, 196), 0)
    p_col = jax.lax.broadcasted_iota(jnp.int32, (14, 196), 1)
    Rt = (p_col // 14 == s_row).astype(jnp.float32)
    Tt = (p_col % 14 == s_row).astype(jnp.float32)

    def sel(slab, onehot):
        return jax.lax.dot_general(
            slab, onehot, (((1,), (0,)), ((), ())),
            preferred_element_type=jnp.float32,
            precision=jax.lax.Precision.HIGHEST,
        )

    i0 = pl.program_id(0) * _IB

    def cslab(ref):
        return ref[pl.ds(i0, _IB)].reshape(16 * _IB, 14)

    Lc1 = sel(cslab(cx1_s), Rt) * sel(cslab(cy1_s), Tt)
    Lc2 = sel(cslab(cx2_s), Rt) * sel(cslab(cy2_s), Tt)

    B1 = b1_s[...]
    B2 = b2_s[...]

    for k in range(_IB):
        i = i0 + k
        ix = jax.lax.div(i, 14)
        iy = jax.lax.rem(i, 14)
        Lz1 = zx1_s[ix] * zy1_s[iy]
        Lz2 = zx2_s[ix] * zy2_s[iy]
        lo, hi = 16 * k, 16 * (k + 1)
        Lc1k = Lc1[lo:hi]
        Lc2k = Lc2[lo:hi]
        A1 = a1_s[i]
        A2 = a2_s[i]

        def emit(o_ref, W, A, B):
            WT = jnp.transpose(W)
            o_ref[k] = (WT[:, :, None] * A[None, :, :]) * B

        emit(o1_ref, Lc1k * Lz1, A1, B1)
        emit(o2_ref, Lc2k * Lz1, A2, B1)
        emit(o3_ref, Lc1k * Lz2, A1, B2)
        emit(o4_ref, Lc2k * Lz2, A2, B2)


def kernel(inference):
    N = inference.shape[0]
    inf = inference.reshape(N, 204, 196)
    f32 = jnp.float32
    scratch = (
        [pltpu.VMEM((196, 16, 20), f32)] * 4
        + [pltpu.VMEM((196, 16, 14), f32)] * 4
        + [pltpu.VMEM((14, 16, 196), f32)] * 4
    )
    outs = pl.pallas_call(
        _fused_body,
        grid=(196 // _IB,),
        in_specs=[pl.BlockSpec((N, 204, 196), lambda i: (0, 0, 0))],
        out_specs=[pl.BlockSpec((_IB, 196, 16, 20), lambda i: (i, 0, 0, 0))] * 4,
        out_shape=[jax.ShapeDtypeStruct((196, 196, 16, 20), f32)] * 4,
        scratch_shapes=scratch,
        compiler_params=pltpu.CompilerParams(
            dimension_semantics=("arbitrary",),
        ),
    )(inf)
    return tuple(
        jnp.transpose(o, (2, 3, 0, 1)).reshape(N, 20, 14, 14, 14, 14) for o in outs
    )
